# baseline (speedup 1.0000x reference)
import functools

import jax
import jax.numpy as jnp
from jax import lax
from jax.experimental import pallas as pl
from jax.experimental.pallas import tpu as pltpu
from jax.experimental.pallas import tpu_sc as plsc

N = 50000
E = 800000
IN_DIM = 58
HID = 100

NC = 2
NS = 16
OPW = 128

PAD_E = 819200
TOT_OPS = PAD_E // OPW
OPS1_TILE = TOT_OPS // NS
OPS2_TILE = TOT_OPS // (NC * NS)
CH1 = 80
CH2 = 40
NB = 10

NPAD = 50048
ROWS_TILE = NPAD // NS
DUMMY = N

_mesh = plsc.VectorSubcoreMesh(core_axis_name="c", subcore_axis_name="s")
_sc_params = pltpu.CompilerParams(use_tc_tiling_on_sc=False)


def _zero_acc(const_hbm, wb_v, acc_sh, s):
    pltpu.sync_copy(const_hbm.at[0], wb_v)

    def body(k, _):
        r0 = s * ROWS_TILE + k * 128
        pltpu.sync_copy(wb_v, acc_sh.at[pl.ds(r0, 128)])
        return 0

    lax.fori_loop(0, ROWS_TILE // 128, body, 0)
    tail = s * ROWS_TILE + (ROWS_TILE // 128) * 128
    pltpu.sync_copy(wb_v.at[pl.ds(0, ROWS_TILE % 128)],
                    acc_sh.at[pl.ds(tail, ROWS_TILE % 128)])


def _write_acc(out_hbm, q, wb_v, acc_sh, s):
    def body(k, _):
        r0 = s * ROWS_TILE + k * 128
        pltpu.sync_copy(acc_sh.at[pl.ds(r0, 128)], wb_v)
        pltpu.sync_copy(wb_v, out_hbm.at[q, pl.ds(r0, 128)])
        return 0

    lax.fori_loop(0, ROWS_TILE // 128, body, 0)
    tail = s * ROWS_TILE + (ROWS_TILE // 128) * 128
    nt = ROWS_TILE % 128
    pltpu.sync_copy(acc_sh.at[pl.ds(tail, nt)], wb_v.at[pl.ds(0, nt)])
    pltpu.sync_copy(wb_v.at[pl.ds(0, nt)], out_hbm.at[q, pl.ds(tail, nt)])


def _ring(tab_hbm, acc_sh, sidx_v, didx_v, bufs, gsems, ssems, n_ops):
    grp = n_ops // NB
    gd = [pltpu.async_copy(tab_hbm.at[sidx_v.at[b]], bufs[b], gsems[b])
          for b in range(NB)]
    sd = [None] * NB
    for g in range(grp):
        for b in range(NB):
            gd[b].wait()
            sd[b] = pltpu.async_copy(
                bufs[b], acc_sh.at[didx_v.at[g * NB + b]], ssems[b], add=True)
        for b in range(NB):
            sd[b].wait()
            if g + 1 < grp:
                gd[b] = pltpu.async_copy(
                    tab_hbm.at[sidx_v.at[(g + 1) * NB + b]], bufs[b], gsems[b])


@functools.partial(
    pl.kernel,
    out_type=jax.ShapeDtypeStruct((NC, NPAD, 16), jnp.float32),
    mesh=_mesh,
    compiler_params=_sc_params,
    scratch_types=[
        pltpu.VMEM((CH2, OPW), jnp.int32),
        pltpu.VMEM((OPW, 16), jnp.float32),
        pltpu.VMEM((128, 16), jnp.float32),
        [pltpu.SemaphoreType.DMA] * NB,
        pltpu.VMEM_SHARED((NPAD, 16), jnp.float32),
    ],
)
def _deg_kernel(dst_hbm, const_hbm, out_hbm, idx_v, ones_v, wb_v, sems, acc_sh):
    c = lax.axis_index("c")
    s = lax.axis_index("s")
    pltpu.sync_copy(const_hbm.at[1], ones_v)
    _zero_acc(const_hbm, wb_v, acc_sh, s)
    plsc.subcore_barrier()

    def stage(st, _):
        row0 = c * (TOT_OPS // NC) + s * OPS2_TILE + st * CH2
        pltpu.sync_copy(dst_hbm.at[pl.ds(row0, CH2)], idx_v)
        for g in range(CH2 // NB):
            descs = []
            for b in range(NB):
                descs.append(pltpu.async_copy(
                    ones_v, acc_sh.at[idx_v.at[g * NB + b]], sems[b],
                    add=True))
            for d in descs:
                d.wait()
        return 0

    lax.fori_loop(0, OPS2_TILE // CH2, stage, 0)
    plsc.subcore_barrier()
    _write_acc(out_hbm, c, wb_v, acc_sh, s)


@functools.partial(
    pl.kernel,
    out_type=jax.ShapeDtypeStruct((4, NPAD, 16), jnp.float32),
    mesh=_mesh,
    compiler_params=_sc_params,
    scratch_types=[
        pltpu.VMEM((CH1, OPW), jnp.int32),
        pltpu.VMEM((CH1, OPW), jnp.int32),
        [pltpu.VMEM((OPW, 16), jnp.float32)] * NB,
        [pltpu.SemaphoreType.DMA] * NB,
        [pltpu.SemaphoreType.DMA] * NB,
        pltpu.VMEM_SHARED((NPAD, 16), jnp.float32),
    ],
)
def _agg1_kernel(src_hbm, dst_hbm, tab4_hbm, const_hbm, out_hbm,
                 sidx_v, didx_v, bufs, gsems, ssems, acc_sh):
    c = lax.axis_index("c")
    s = lax.axis_index("s")
    for p in range(2):
        q = 2 * c + p
        _zero_acc(const_hbm, bufs[0], acc_sh, s)
        plsc.subcore_barrier()

        def stage(st, _):
            base = s * OPS1_TILE + st * CH1 + c * (TOT_OPS // 2)
            row0 = lax.rem(base, TOT_OPS)
            pltpu.sync_copy(src_hbm.at[pl.ds(row0, CH1)], sidx_v)
            pltpu.sync_copy(dst_hbm.at[pl.ds(row0, CH1)], didx_v)
            _ring(tab4_hbm.at[q], acc_sh, sidx_v, didx_v, bufs, gsems, ssems,
                  CH1)
            return 0

        lax.fori_loop(0, OPS1_TILE // CH1, stage, 0)
        plsc.subcore_barrier()
        _write_acc(out_hbm, q, bufs[0], acc_sh, s)
        if p == 0:
            plsc.subcore_barrier()


@functools.partial(
    pl.kernel,
    out_type=jax.ShapeDtypeStruct((NC, NPAD, 16), jnp.float32),
    mesh=_mesh,
    compiler_params=_sc_params,
    scratch_types=[
        pltpu.VMEM((CH2, OPW), jnp.int32),
        pltpu.VMEM((CH2, OPW), jnp.int32),
        [pltpu.VMEM((OPW, 16), jnp.float32)] * NB,
        [pltpu.SemaphoreType.DMA] * NB,
        [pltpu.SemaphoreType.DMA] * NB,
        pltpu.VMEM_SHARED((NPAD, 16), jnp.float32),
    ],
)
def _agg2_kernel(src_hbm, dst_hbm, tab_hbm, const_hbm, out_hbm,
                 sidx_v, didx_v, bufs, gsems, ssems, acc_sh):
    c = lax.axis_index("c")
    s = lax.axis_index("s")
    _zero_acc(const_hbm, bufs[0], acc_sh, s)
    plsc.subcore_barrier()

    def stage(st, _):
        row0 = c * (TOT_OPS // NC) + s * OPS2_TILE + st * CH2
        pltpu.sync_copy(src_hbm.at[pl.ds(row0, CH2)], sidx_v)
        pltpu.sync_copy(dst_hbm.at[pl.ds(row0, CH2)], didx_v)
        _ring(tab_hbm, acc_sh, sidx_v, didx_v, bufs, gsems, ssems, CH2)
        return 0

    lax.fori_loop(0, OPS2_TILE // CH2, stage, 0)
    plsc.subcore_barrier()
    _write_acc(out_hbm, c, bufs[0], acc_sh, s)


def _dense0_body(dacc_ref, x_ref, y_ref, dinv_ref):
    deg = dacc_ref[0][:, 0:1] + dacc_ref[1][:, 0:1] + 1.0
    dinv = lax.rsqrt(deg)
    y = x_ref[...] * dinv
    for q in range(4):
        y_ref[q] = y[:, 16 * q:16 * (q + 1)]
    dinv_ref[...] = dinv


def _dense1_body(zacc_ref, y_ref, dinv_ref, w1_ref, b1_ref, w2_ref, s_ref):
    z = jnp.concatenate(
        [zacc_ref[0], zacc_ref[1], zacc_ref[2], zacc_ref[3]], axis=1)
    y = jnp.concatenate([y_ref[0], y_ref[1], y_ref[2], y_ref[3]], axis=1)
    agg = (z + y) * dinv_ref[...]
    h = jnp.dot(agg, w1_ref[...], preferred_element_type=jnp.float32)
    h = jnp.maximum(h + b1_ref[...], 0.0)
    sp = jnp.dot(h, w2_ref[...], preferred_element_type=jnp.float32)
    sp = sp * dinv_ref[...]
    s_ref[...] = jnp.concatenate(
        [sp, jnp.zeros((sp.shape[0], 15), jnp.float32)], axis=1)


def _dense2_body(z2acc_ref, s16_ref, dinv_ref, b2_ref, out_ref):
    z2 = z2acc_ref[0][:, 0:1] + z2acc_ref[1][:, 0:1]
    out_ref[...] = dinv_ref[...] * (z2 + s16_ref[...][:, 0:1]) + b2_ref[...]


def kernel(x, edge_index, W1, b1, W2, b2):
    f32 = jnp.float32
    src = edge_index[0].astype(jnp.int32)
    dst = edge_index[1].astype(jnp.int32)
    pad = PAD_E - E
    srcp = jnp.concatenate([src, jnp.zeros((pad,), jnp.int32)])
    dstp = jnp.concatenate([dst, jnp.full((pad,), DUMMY, jnp.int32)])
    src_ops = srcp.reshape(TOT_OPS, OPW)
    dst_ops = dstp.reshape(TOT_OPS, OPW)
    x64 = jnp.pad(x, ((0, 0), (0, 64 - IN_DIM)))
    const16 = jnp.stack([jnp.zeros((128, 16), f32), jnp.ones((128, 16), f32)])
    w1p = jnp.pad(W1, ((0, 64 - IN_DIM), (0, 0)))
    b1r = b1.reshape(1, HID)
    b2r = b2.reshape(1, 1)

    dacc = _deg_kernel(dst_ops, const16)

    r0 = 2000
    y4, dinv = pl.pallas_call(
        _dense0_body,
        grid=(N // r0,),
        in_specs=[
            pl.BlockSpec((NC, r0, 16), lambda i: (0, i, 0)),
            pl.BlockSpec((r0, 64), lambda i: (i, 0)),
        ],
        out_specs=[
            pl.BlockSpec((4, r0, 16), lambda i: (0, i, 0)),
            pl.BlockSpec((r0, 1), lambda i: (i, 0)),
        ],
        out_shape=[
            jax.ShapeDtypeStruct((4, N, 16), f32),
            jax.ShapeDtypeStruct((N, 1), f32),
        ],
    )(dacc, x64)

    zacc = _agg1_kernel(src_ops, dst_ops, y4, const16)

    r1 = 2000
    s16 = pl.pallas_call(
        _dense1_body,
        grid=(N // r1,),
        in_specs=[
            pl.BlockSpec((4, r1, 16), lambda i: (0, i, 0)),
            pl.BlockSpec((4, r1, 16), lambda i: (0, i, 0)),
            pl.BlockSpec((r1, 1), lambda i: (i, 0)),
            pl.BlockSpec((64, HID), lambda i: (0, 0)),
            pl.BlockSpec((1, HID), lambda i: (0, 0)),
            pl.BlockSpec((HID, 1), lambda i: (0, 0)),
        ],
        out_specs=pl.BlockSpec((r1, 16), lambda i: (i, 0)),
        out_shape=jax.ShapeDtypeStruct((N, 16), f32),
    )(zacc, y4, dinv, w1p, b1r, W2)

    z2acc = _agg2_kernel(src_ops, dst_ops, s16, const16)

    r2 = 2000
    out = pl.pallas_call(
        _dense2_body,
        grid=(N // r2,),
        in_specs=[
            pl.BlockSpec((NC, r2, 16), lambda i: (0, i, 0)),
            pl.BlockSpec((r2, 16), lambda i: (i, 0)),
            pl.BlockSpec((r2, 1), lambda i: (i, 0)),
            pl.BlockSpec((1, 1), lambda i: (0, 0)),
        ],
        out_specs=pl.BlockSpec((r2, 1), lambda i: (i, 0)),
        out_shape=jax.ShapeDtypeStruct((N, 1), f32),
    )(z2acc, s16, dinv, b2r)
    return out


# speedup vs baseline: 1.0470x; 1.0470x over previous
import functools

import jax
import jax.numpy as jnp
from jax import lax
from jax.experimental import pallas as pl
from jax.experimental.pallas import tpu as pltpu
from jax.experimental.pallas import tpu_sc as plsc

N = 50000
E = 800000
IN_DIM = 58
HID = 100

NC = 2
NS = 16
OPW = 128

PAD_E = 819200
TOT_OPS = PAD_E // OPW
OPS1_TILE = TOT_OPS // NS
OPS2_TILE = TOT_OPS // (NC * NS)
CH1 = 80
CH2 = 40
NB = 10

NPAD = 52096
ROWS_TILE = NPAD // NS
NGARB = NPAD - N

_mesh = plsc.VectorSubcoreMesh(core_axis_name="c", subcore_axis_name="s")
_sc_params = pltpu.CompilerParams(use_tc_tiling_on_sc=False)


def _zero_acc(const_hbm, wb_v, acc_sh, s):
    pltpu.sync_copy(const_hbm.at[0], wb_v)

    def body(k, _):
        r0 = s * ROWS_TILE + k * 128
        pltpu.sync_copy(wb_v, acc_sh.at[pl.ds(r0, 128)])
        return 0

    lax.fori_loop(0, ROWS_TILE // 128, body, 0)
    tail = s * ROWS_TILE + (ROWS_TILE // 128) * 128
    pltpu.sync_copy(wb_v.at[pl.ds(0, ROWS_TILE % 128)],
                    acc_sh.at[pl.ds(tail, ROWS_TILE % 128)])


def _write_acc(out_hbm, q, wb_v, acc_sh, s):
    def body(k, _):
        r0 = s * ROWS_TILE + k * 128
        pltpu.sync_copy(acc_sh.at[pl.ds(r0, 128)], wb_v)
        pltpu.sync_copy(wb_v, out_hbm.at[q, pl.ds(r0, 128)])
        return 0

    lax.fori_loop(0, ROWS_TILE // 128, body, 0)
    tail = s * ROWS_TILE + (ROWS_TILE // 128) * 128
    nt = ROWS_TILE % 128
    pltpu.sync_copy(acc_sh.at[pl.ds(tail, nt)], wb_v.at[pl.ds(0, nt)])
    pltpu.sync_copy(wb_v.at[pl.ds(0, nt)], out_hbm.at[q, pl.ds(tail, nt)])


def _ring(tab_hbm, acc_sh, sidx_v, didx_v, bufs, gsems, ssems, n_ops):
    grp = n_ops // NB
    gd = [pltpu.async_copy(tab_hbm.at[sidx_v.at[b]], bufs[b], gsems[b])
          for b in range(NB)]
    sd = [None] * NB
    for g in range(grp):
        for b in range(NB):
            gd[b].wait()
            sd[b] = pltpu.async_copy(
                bufs[b], acc_sh.at[didx_v.at[g * NB + b]], ssems[b], add=True)
        for b in range(NB):
            sd[b].wait()
            if g + 1 < grp:
                gd[b] = pltpu.async_copy(
                    tab_hbm.at[sidx_v.at[(g + 1) * NB + b]], bufs[b], gsems[b])


@functools.partial(
    pl.kernel,
    out_type=jax.ShapeDtypeStruct((NC, NPAD, 16), jnp.float32),
    mesh=_mesh,
    compiler_params=_sc_params,
    scratch_types=[
        pltpu.VMEM((CH2, OPW), jnp.int32),
        pltpu.VMEM((OPW, 16), jnp.float32),
        pltpu.VMEM((128, 16), jnp.float32),
        [pltpu.SemaphoreType.DMA] * NB,
        pltpu.VMEM_SHARED((NPAD, 16), jnp.float32),
    ],
)
def _deg_kernel(dst_hbm, const_hbm, out_hbm, idx_v, ones_v, wb_v, sems, acc_sh):
    c = lax.axis_index("c")
    s = lax.axis_index("s")
    pltpu.sync_copy(const_hbm.at[1], ones_v)
    _zero_acc(const_hbm, wb_v, acc_sh, s)
    plsc.subcore_barrier()

    def stage(st, _):
        row0 = c * (TOT_OPS // NC) + s * OPS2_TILE + st * CH2
        pltpu.sync_copy(dst_hbm.at[pl.ds(row0, CH2)], idx_v)
        for g in range(CH2 // NB):
            descs = []
            for b in range(NB):
                descs.append(pltpu.async_copy(
                    ones_v, acc_sh.at[idx_v.at[g * NB + b]], sems[b],
                    add=True))
            for d in descs:
                d.wait()
        return 0

    lax.fori_loop(0, OPS2_TILE // CH2, stage, 0)
    plsc.subcore_barrier()
    _write_acc(out_hbm, c, wb_v, acc_sh, s)


@functools.partial(
    pl.kernel,
    out_type=jax.ShapeDtypeStruct((4, NPAD, 16), jnp.float32),
    mesh=_mesh,
    compiler_params=_sc_params,
    scratch_types=[
        pltpu.VMEM((CH1, OPW), jnp.int32),
        pltpu.VMEM((CH1, OPW), jnp.int32),
        [pltpu.VMEM((OPW, 16), jnp.float32)] * NB,
        [pltpu.SemaphoreType.DMA] * NB,
        [pltpu.SemaphoreType.DMA] * NB,
        pltpu.VMEM_SHARED((NPAD, 16), jnp.float32),
    ],
)
def _agg1_kernel(src_hbm, dst_hbm, tab4_hbm, const_hbm, out_hbm,
                 sidx_v, didx_v, bufs, gsems, ssems, acc_sh):
    c = lax.axis_index("c")
    s = lax.axis_index("s")
    for p in range(2):
        q = 2 * c + p
        _zero_acc(const_hbm, bufs[0], acc_sh, s)
        plsc.subcore_barrier()

        def stage(st, _):
            row0 = s * OPS1_TILE + st * CH1
            pltpu.sync_copy(src_hbm.at[pl.ds(row0, CH1)], sidx_v)
            pltpu.sync_copy(dst_hbm.at[pl.ds(row0, CH1)], didx_v)
            _ring(tab4_hbm.at[q], acc_sh, sidx_v, didx_v, bufs, gsems, ssems,
                  CH1)
            return 0

        lax.fori_loop(0, OPS1_TILE // CH1, stage, 0)
        plsc.subcore_barrier()
        _write_acc(out_hbm, q, bufs[0], acc_sh, s)
        if p == 0:
            plsc.subcore_barrier()


@functools.partial(
    pl.kernel,
    out_type=jax.ShapeDtypeStruct((NC, NPAD, 16), jnp.float32),
    mesh=_mesh,
    compiler_params=_sc_params,
    scratch_types=[
        pltpu.VMEM((CH2, OPW), jnp.int32),
        pltpu.VMEM((CH2, OPW), jnp.int32),
        [pltpu.VMEM((OPW, 16), jnp.float32)] * NB,
        [pltpu.SemaphoreType.DMA] * NB,
        [pltpu.SemaphoreType.DMA] * NB,
        pltpu.VMEM_SHARED((NPAD, 16), jnp.float32),
    ],
)
def _agg2_kernel(src_hbm, dst_hbm, tab_hbm, const_hbm, out_hbm,
                 sidx_v, didx_v, bufs, gsems, ssems, acc_sh):
    c = lax.axis_index("c")
    s = lax.axis_index("s")
    _zero_acc(const_hbm, bufs[0], acc_sh, s)
    plsc.subcore_barrier()

    def stage(st, _):
        row0 = c * (TOT_OPS // NC) + s * OPS2_TILE + st * CH2
        pltpu.sync_copy(src_hbm.at[pl.ds(row0, CH2)], sidx_v)
        pltpu.sync_copy(dst_hbm.at[pl.ds(row0, CH2)], didx_v)
        _ring(tab_hbm, acc_sh, sidx_v, didx_v, bufs, gsems, ssems, CH2)
        return 0

    lax.fori_loop(0, OPS2_TILE // CH2, stage, 0)
    plsc.subcore_barrier()
    _write_acc(out_hbm, c, bufs[0], acc_sh, s)


def _dense0_body(dacc_ref, x_ref, y_ref, dinv_ref):
    deg = dacc_ref[0][:, 0:1] + dacc_ref[1][:, 0:1] + 1.0
    dinv = lax.rsqrt(deg)
    y = x_ref[...] * dinv
    for q in range(4):
        y_ref[q] = y[:, 16 * q:16 * (q + 1)]
    dinv_ref[...] = dinv


def _dense1_body(zacc_ref, y_ref, dinv_ref, w1_ref, b1_ref, w2_ref, s_ref):
    z = jnp.concatenate(
        [zacc_ref[0], zacc_ref[1], zacc_ref[2], zacc_ref[3]], axis=1)
    y = jnp.concatenate([y_ref[0], y_ref[1], y_ref[2], y_ref[3]], axis=1)
    agg = (z + y) * dinv_ref[...]
    h = jnp.dot(agg, w1_ref[...], preferred_element_type=jnp.float32)
    h = jnp.maximum(h + b1_ref[...], 0.0)
    sp = jnp.dot(h, w2_ref[...], preferred_element_type=jnp.float32)
    sp = sp * dinv_ref[...]
    s_ref[...] = jnp.concatenate(
        [sp, jnp.zeros((sp.shape[0], 15), jnp.float32)], axis=1)


def _dense2_body(z2acc_ref, s16_ref, dinv_ref, b2_ref, out_ref):
    z2 = z2acc_ref[0][:, 0:1] + z2acc_ref[1][:, 0:1]
    out_ref[...] = dinv_ref[...] * (z2 + s16_ref[...][:, 0:1]) + b2_ref[...]


def kernel(x, edge_index, W1, b1, W2, b2):
    f32 = jnp.float32
    src = edge_index[0].astype(jnp.int32)
    dst = edge_index[1].astype(jnp.int32)
    pad = PAD_E - E
    srcp = jnp.concatenate([src, jnp.zeros((pad,), jnp.int32)])
    garb = N + jnp.arange(pad, dtype=jnp.int32) % NGARB
    dstp = jnp.concatenate([dst, garb])
    src_ops = srcp.reshape(TOT_OPS, OPW)
    dst_ops = dstp.reshape(TOT_OPS, OPW)
    x64 = jnp.pad(x, ((0, 0), (0, 64 - IN_DIM)))
    const16 = jnp.stack([jnp.zeros((128, 16), f32), jnp.ones((128, 16), f32)])
    w1p = jnp.pad(W1, ((0, 64 - IN_DIM), (0, 0)))
    b1r = b1.reshape(1, HID)
    b2r = b2.reshape(1, 1)

    dacc = _deg_kernel(dst_ops, const16)

    r0 = 2000
    y4, dinv = pl.pallas_call(
        _dense0_body,
        grid=(N // r0,),
        in_specs=[
            pl.BlockSpec((NC, r0, 16), lambda i: (0, i, 0)),
            pl.BlockSpec((r0, 64), lambda i: (i, 0)),
        ],
        out_specs=[
            pl.BlockSpec((4, r0, 16), lambda i: (0, i, 0)),
            pl.BlockSpec((r0, 1), lambda i: (i, 0)),
        ],
        out_shape=[
            jax.ShapeDtypeStruct((4, N, 16), f32),
            jax.ShapeDtypeStruct((N, 1), f32),
        ],
    )(dacc, x64)

    zacc = _agg1_kernel(src_ops, dst_ops, y4, const16)

    r1 = 2000
    s16 = pl.pallas_call(
        _dense1_body,
        grid=(N // r1,),
        in_specs=[
            pl.BlockSpec((4, r1, 16), lambda i: (0, i, 0)),
            pl.BlockSpec((4, r1, 16), lambda i: (0, i, 0)),
            pl.BlockSpec((r1, 1), lambda i: (i, 0)),
            pl.BlockSpec((64, HID), lambda i: (0, 0)),
            pl.BlockSpec((1, HID), lambda i: (0, 0)),
            pl.BlockSpec((HID, 1), lambda i: (0, 0)),
        ],
        out_specs=pl.BlockSpec((r1, 16), lambda i: (i, 0)),
        out_shape=jax.ShapeDtypeStruct((N, 16), f32),
    )(zacc, y4, dinv, w1p, b1r, W2)

    z2acc = _agg2_kernel(src_ops, dst_ops, s16, const16)

    r2 = 2000
    out = pl.pallas_call(
        _dense2_body,
        grid=(N // r2,),
        in_specs=[
            pl.BlockSpec((NC, r2, 16), lambda i: (0, i, 0)),
            pl.BlockSpec((r2, 16), lambda i: (i, 0)),
            pl.BlockSpec((r2, 1), lambda i: (i, 0)),
            pl.BlockSpec((1, 1), lambda i: (0, 0)),
        ],
        out_specs=pl.BlockSpec((r2, 1), lambda i: (i, 0)),
        out_shape=jax.ShapeDtypeStruct((N, 1), f32),
    )(z2acc, s16, dinv, b2r)
    return out


# speedup vs baseline: 1.6539x; 1.5797x over previous
import functools

import jax
import jax.numpy as jnp
from jax import lax
from jax.experimental import pallas as pl
from jax.experimental.pallas import tpu as pltpu
from jax.experimental.pallas import tpu_sc as plsc

N = 50000
E = 800000
IN_DIM = 58
HID = 100

NC = 2
NS = 16
OPW = 128

PAD_E = 819200
TOT_OPS = PAD_E // OPW
OPS1_TILE = TOT_OPS // NS
OPS2_TILE = TOT_OPS // (NC * NS)
CH1 = 80
CH2 = 40
NB = 10

NPAD = 52096
ROWS_TILE = NPAD // NS
NGARB = NPAD - N

_mesh = plsc.VectorSubcoreMesh(core_axis_name="c", subcore_axis_name="s")
_sc_params = pltpu.CompilerParams(use_tc_tiling_on_sc=False)


def _zero_acc(const_hbm, wb_v, acc_sh, s):
    pltpu.sync_copy(const_hbm.at[0], wb_v)

    def body(k, _):
        r0 = s * ROWS_TILE + k * 128
        pltpu.sync_copy(wb_v, acc_sh.at[pl.ds(r0, 128)])
        return 0

    lax.fori_loop(0, ROWS_TILE // 128, body, 0)
    tail = s * ROWS_TILE + (ROWS_TILE // 128) * 128
    pltpu.sync_copy(wb_v.at[pl.ds(0, ROWS_TILE % 128)],
                    acc_sh.at[pl.ds(tail, ROWS_TILE % 128)])


def _write_acc(out_hbm, q, wb_v, acc_sh, s):
    def body(k, _):
        r0 = s * ROWS_TILE + k * 128
        pltpu.sync_copy(acc_sh.at[pl.ds(r0, 128)], wb_v)
        pltpu.sync_copy(wb_v, out_hbm.at[q, pl.ds(r0, 128)])
        return 0

    lax.fori_loop(0, ROWS_TILE // 128, body, 0)
    tail = s * ROWS_TILE + (ROWS_TILE // 128) * 128
    nt = ROWS_TILE % 128
    pltpu.sync_copy(acc_sh.at[pl.ds(tail, nt)], wb_v.at[pl.ds(0, nt)])
    pltpu.sync_copy(wb_v.at[pl.ds(0, nt)], out_hbm.at[q, pl.ds(tail, nt)])


def _ring(tab_hbm, acc_sh, sidx_v, didx_v, bufs, gsems, ssems, n_ops):
    grp = n_ops // NB
    gd = [pltpu.async_copy(tab_hbm.at[sidx_v.at[b]], bufs[b], gsems[b])
          for b in range(NB)]
    sd = [None] * NB
    for g in range(grp):
        for b in range(NB):
            gd[b].wait()
            sd[b] = pltpu.async_copy(
                bufs[b], acc_sh.at[didx_v.at[g * NB + b]], ssems[b], add=True)
        for b in range(NB):
            sd[b].wait()
            if g + 1 < grp:
                gd[b] = pltpu.async_copy(
                    tab_hbm.at[sidx_v.at[(g + 1) * NB + b]], bufs[b], gsems[b])


@functools.partial(
    pl.kernel,
    out_type=jax.ShapeDtypeStruct((NC, NPAD, 16), jnp.float32),
    mesh=_mesh,
    compiler_params=_sc_params,
    scratch_types=[
        pltpu.VMEM((CH2, OPW), jnp.int32),
        pltpu.VMEM((OPW, 16), jnp.float32),
        pltpu.VMEM((128, 16), jnp.float32),
        [pltpu.SemaphoreType.DMA] * NB,
        pltpu.VMEM_SHARED((NPAD, 16), jnp.float32),
    ],
)
def _deg_kernel(dst_hbm, const_hbm, out_hbm, idx_v, ones_v, wb_v, sems, acc_sh):
    c = lax.axis_index("c")
    s = lax.axis_index("s")
    pltpu.sync_copy(const_hbm.at[1], ones_v)
    _zero_acc(const_hbm, wb_v, acc_sh, s)
    plsc.subcore_barrier()

    def stage(st, _):
        row0 = c * (TOT_OPS // NC) + s * OPS2_TILE + st * CH2
        pltpu.sync_copy(dst_hbm.at[pl.ds(row0, CH2)], idx_v)
        for g in range(CH2 // NB):
            descs = []
            for b in range(NB):
                descs.append(pltpu.async_copy(
                    ones_v, acc_sh.at[idx_v.at[g * NB + b]], sems[b],
                    add=True))
            for d in descs:
                d.wait()
        return 0

    lax.fori_loop(0, OPS2_TILE // CH2, stage, 0)
    plsc.subcore_barrier()
    _write_acc(out_hbm, c, wb_v, acc_sh, s)


def _write_acc_cols(out_hbm, q, wb_v, acc_sh, s):
    def body(k, _):
        r0 = s * ROWS_TILE + k * 128
        pltpu.sync_copy(acc_sh.at[pl.ds(r0, 128)], wb_v)
        pltpu.sync_copy(wb_v, out_hbm.at[pl.ds(r0, 128), pl.ds(16 * q, 16)])
        return 0

    lax.fori_loop(0, ROWS_TILE // 128, body, 0)
    tail = s * ROWS_TILE + (ROWS_TILE // 128) * 128
    nt = ROWS_TILE % 128
    pltpu.sync_copy(acc_sh.at[pl.ds(tail, nt)], wb_v.at[pl.ds(0, nt)])
    pltpu.sync_copy(wb_v.at[pl.ds(0, nt)],
                    out_hbm.at[pl.ds(tail, nt), pl.ds(16 * q, 16)])


@functools.partial(
    pl.kernel,
    out_type=jax.ShapeDtypeStruct((NPAD, 64), jnp.float32),
    mesh=_mesh,
    compiler_params=_sc_params,
    scratch_types=[
        pltpu.VMEM((CH1, OPW), jnp.int32),
        pltpu.VMEM((CH1, OPW), jnp.int32),
        [pltpu.VMEM((OPW, 16), jnp.float32)] * NB,
        [pltpu.SemaphoreType.DMA] * NB,
        [pltpu.SemaphoreType.DMA] * NB,
        pltpu.VMEM_SHARED((NPAD, 16), jnp.float32),
    ],
)
def _agg1_kernel(src_hbm, dst_hbm, tab4_hbm, const_hbm, out_hbm,
                 sidx_v, didx_v, bufs, gsems, ssems, acc_sh):
    c = lax.axis_index("c")
    s = lax.axis_index("s")
    for p in range(2):
        q = 2 * c + p
        _zero_acc(const_hbm, bufs[0], acc_sh, s)
        plsc.subcore_barrier()

        def stage(st, _):
            row0 = s * OPS1_TILE + st * CH1
            pltpu.sync_copy(src_hbm.at[pl.ds(row0, CH1)], sidx_v)
            pltpu.sync_copy(dst_hbm.at[pl.ds(row0, CH1)], didx_v)
            _ring(tab4_hbm.at[q], acc_sh, sidx_v, didx_v, bufs, gsems, ssems,
                  CH1)
            return 0

        lax.fori_loop(0, OPS1_TILE // CH1, stage, 0)
        plsc.subcore_barrier()
        _write_acc_cols(out_hbm, q, bufs[0], acc_sh, s)
        if p == 0:
            plsc.subcore_barrier()


@functools.partial(
    pl.kernel,
    out_type=jax.ShapeDtypeStruct((NC, NPAD, 16), jnp.float32),
    mesh=_mesh,
    compiler_params=_sc_params,
    scratch_types=[
        pltpu.VMEM((CH2, OPW), jnp.int32),
        pltpu.VMEM((CH2, OPW), jnp.int32),
        [pltpu.VMEM((OPW, 16), jnp.float32)] * NB,
        [pltpu.SemaphoreType.DMA] * NB,
        [pltpu.SemaphoreType.DMA] * NB,
        pltpu.VMEM_SHARED((NPAD, 16), jnp.float32),
    ],
)
def _agg2_kernel(src_hbm, dst_hbm, tab_hbm, const_hbm, out_hbm,
                 sidx_v, didx_v, bufs, gsems, ssems, acc_sh):
    c = lax.axis_index("c")
    s = lax.axis_index("s")
    _zero_acc(const_hbm, bufs[0], acc_sh, s)
    plsc.subcore_barrier()

    def stage(st, _):
        row0 = c * (TOT_OPS // NC) + s * OPS2_TILE + st * CH2
        pltpu.sync_copy(src_hbm.at[pl.ds(row0, CH2)], sidx_v)
        pltpu.sync_copy(dst_hbm.at[pl.ds(row0, CH2)], didx_v)
        _ring(tab_hbm, acc_sh, sidx_v, didx_v, bufs, gsems, ssems, CH2)
        return 0

    lax.fori_loop(0, OPS2_TILE // CH2, stage, 0)
    plsc.subcore_barrier()
    _write_acc(out_hbm, c, bufs[0], acc_sh, s)


def _dense0_body(dacc_ref, x_ref, y_ref, dinv_ref):
    deg = dacc_ref[0][:, 0:1] + dacc_ref[1][:, 0:1] + 1.0
    dinv = lax.rsqrt(deg)
    y = x_ref[...] * dinv
    for q in range(4):
        y_ref[q] = y[:, 16 * q:16 * (q + 1)]
    dinv_ref[...] = dinv


def _dense1_body(z_ref, y_ref, dinv_ref, w1_ref, b1_ref, w2_ref, s_ref):
    y = jnp.concatenate([y_ref[0], y_ref[1], y_ref[2], y_ref[3]], axis=1)
    agg = (z_ref[...] + y) * dinv_ref[...]
    h = jnp.dot(agg, w1_ref[...], preferred_element_type=jnp.float32)
    h = jnp.maximum(h + b1_ref[...], 0.0)
    sp = jnp.dot(h, w2_ref[...], preferred_element_type=jnp.float32)
    sp = sp * dinv_ref[...]
    s_ref[...] = jnp.concatenate(
        [sp, jnp.zeros((sp.shape[0], 15), jnp.float32)], axis=1)


def _dense2_body(z2acc_ref, s16_ref, dinv_ref, b2_ref, out_ref):
    z2 = z2acc_ref[0][:, 0:1] + z2acc_ref[1][:, 0:1]
    out_ref[...] = dinv_ref[...] * (z2 + s16_ref[...][:, 0:1]) + b2_ref[...]


def kernel(x, edge_index, W1, b1, W2, b2):
    f32 = jnp.float32
    src = edge_index[0].astype(jnp.int32)
    dst = edge_index[1].astype(jnp.int32)
    pad = PAD_E - E
    spread = jnp.arange(pad, dtype=jnp.int32)
    srcp = jnp.concatenate([src, spread % N])
    dstp = jnp.concatenate([dst, N + spread % NGARB])
    src_ops = srcp.reshape(TOT_OPS, OPW)
    dst_ops = dstp.reshape(TOT_OPS, OPW)
    x64 = jnp.pad(x, ((0, 0), (0, 64 - IN_DIM)))
    const16 = jnp.stack([jnp.zeros((128, 16), f32), jnp.ones((128, 16), f32)])
    w1p = jnp.pad(W1, ((0, 64 - IN_DIM), (0, 0)))
    b1r = b1.reshape(1, HID)
    b2r = b2.reshape(1, 1)

    dacc = _deg_kernel(dst_ops, const16)

    r0 = 2000
    y4, dinv = pl.pallas_call(
        _dense0_body,
        grid=(N // r0,),
        in_specs=[
            pl.BlockSpec((NC, r0, 16), lambda i: (0, i, 0)),
            pl.BlockSpec((r0, 64), lambda i: (i, 0)),
        ],
        out_specs=[
            pl.BlockSpec((4, r0, 16), lambda i: (0, i, 0)),
            pl.BlockSpec((r0, 1), lambda i: (i, 0)),
        ],
        out_shape=[
            jax.ShapeDtypeStruct((4, N, 16), f32),
            jax.ShapeDtypeStruct((N, 1), f32),
        ],
    )(dacc, x64)

    zacc = _agg1_kernel(src_ops, dst_ops, y4, const16)

    r1 = 2000
    s16 = pl.pallas_call(
        _dense1_body,
        grid=(N // r1,),
        in_specs=[
            pl.BlockSpec((r1, 64), lambda i: (i, 0)),
            pl.BlockSpec((4, r1, 16), lambda i: (0, i, 0)),
            pl.BlockSpec((r1, 1), lambda i: (i, 0)),
            pl.BlockSpec((64, HID), lambda i: (0, 0)),
            pl.BlockSpec((1, HID), lambda i: (0, 0)),
            pl.BlockSpec((HID, 1), lambda i: (0, 0)),
        ],
        out_specs=pl.BlockSpec((r1, 16), lambda i: (i, 0)),
        out_shape=jax.ShapeDtypeStruct((N, 16), f32),
    )(zacc, y4, dinv, w1p, b1r, W2)

    z2acc = _agg2_kernel(src_ops, dst_ops, s16, const16)

    r2 = 2000
    out = pl.pallas_call(
        _dense2_body,
        grid=(N // r2,),
        in_specs=[
            pl.BlockSpec((NC, r2, 16), lambda i: (0, i, 0)),
            pl.BlockSpec((r2, 16), lambda i: (i, 0)),
            pl.BlockSpec((r2, 1), lambda i: (i, 0)),
            pl.BlockSpec((1, 1), lambda i: (0, 0)),
        ],
        out_specs=pl.BlockSpec((r2, 1), lambda i: (i, 0)),
        out_shape=jax.ShapeDtypeStruct((N, 1), f32),
    )(z2acc, s16, dinv, b2r)
    return out


# speedup vs baseline: 1.9764x; 1.1950x over previous
import functools

import jax
import jax.numpy as jnp
from jax import lax
from jax.experimental import pallas as pl
from jax.experimental.pallas import tpu as pltpu
from jax.experimental.pallas import tpu_sc as plsc

N = 50000
E = 800000
IN_DIM = 58
HID = 100

NC = 2
NS = 16
OPW = 128

PAD_E = 819200
TOT_OPS = PAD_E // OPW
OPS1_TILE = TOT_OPS // NS
OPS2_TILE = TOT_OPS // (NC * NS)
CH1 = 80
CH2 = 40
NB = 10

NPAD = 52096
ROWS_TILE = NPAD // NS
NGARB = NPAD - N

_mesh = plsc.VectorSubcoreMesh(core_axis_name="c", subcore_axis_name="s")
_sc_params = pltpu.CompilerParams(use_tc_tiling_on_sc=False)


def _zero_acc(const_hbm, wb_v, acc_sh, s):
    pltpu.sync_copy(const_hbm.at[0], wb_v)

    def body(k, _):
        r0 = s * ROWS_TILE + k * 128
        pltpu.sync_copy(wb_v, acc_sh.at[pl.ds(r0, 128)])
        return 0

    lax.fori_loop(0, ROWS_TILE // 128, body, 0)
    tail = s * ROWS_TILE + (ROWS_TILE // 128) * 128
    pltpu.sync_copy(wb_v.at[pl.ds(0, ROWS_TILE % 128)],
                    acc_sh.at[pl.ds(tail, ROWS_TILE % 128)])


def _write_acc_cols(out_hbm, q, wb_v, acc_sh, s):
    def body(k, _):
        r0 = s * ROWS_TILE + k * 128
        pltpu.sync_copy(acc_sh.at[pl.ds(r0, 128)], wb_v)
        pltpu.sync_copy(wb_v, out_hbm.at[pl.ds(r0, 128), pl.ds(16 * q, 16)])
        return 0

    lax.fori_loop(0, ROWS_TILE // 128, body, 0)
    tail = s * ROWS_TILE + (ROWS_TILE // 128) * 128
    nt = ROWS_TILE % 128
    pltpu.sync_copy(acc_sh.at[pl.ds(tail, nt)], wb_v.at[pl.ds(0, nt)])
    pltpu.sync_copy(wb_v.at[pl.ds(0, nt)],
                    out_hbm.at[pl.ds(tail, nt), pl.ds(16 * q, 16)])


def _ring(tab_hbm, acc_sh, sidx_v, didx_v, bufs, gsems, ssems, n_ops):
    grp = n_ops // NB
    gd = [pltpu.async_copy(tab_hbm.at[sidx_v.at[b]], bufs[b], gsems[b])
          for b in range(NB)]
    sd = [None] * NB
    for g in range(grp):
        for b in range(NB):
            gd[b].wait()
            sd[b] = pltpu.async_copy(
                bufs[b], acc_sh.at[didx_v.at[g * NB + b]], ssems[b], add=True)
        for b in range(NB):
            sd[b].wait()
            if g + 1 < grp:
                gd[b] = pltpu.async_copy(
                    tab_hbm.at[sidx_v.at[(g + 1) * NB + b]], bufs[b], gsems[b])


@functools.partial(
    pl.kernel,
    out_type=jax.ShapeDtypeStruct((NPAD, 32), jnp.float32),
    mesh=_mesh,
    compiler_params=_sc_params,
    scratch_types=[
        pltpu.VMEM((CH2, OPW), jnp.int32),
        pltpu.VMEM((OPW, 16), jnp.float32),
        pltpu.VMEM((128, 16), jnp.float32),
        [pltpu.SemaphoreType.DMA] * NB,
        pltpu.VMEM_SHARED((NPAD, 16), jnp.float32),
    ],
)
def _deg_kernel(dst_hbm, const_hbm, out_hbm, idx_v, ones_v, wb_v, sems, acc_sh):
    c = lax.axis_index("c")
    s = lax.axis_index("s")
    pltpu.sync_copy(const_hbm.at[1], ones_v)
    _zero_acc(const_hbm, wb_v, acc_sh, s)
    plsc.subcore_barrier()

    def stage(st, _):
        row0 = c * (TOT_OPS // NC) + s * OPS2_TILE + st * CH2
        pltpu.sync_copy(dst_hbm.at[pl.ds(row0, CH2)], idx_v)
        for g in range(CH2 // NB):
            descs = []
            for b in range(NB):
                descs.append(pltpu.async_copy(
                    ones_v, acc_sh.at[idx_v.at[g * NB + b]], sems[b],
                    add=True))
            for d in descs:
                d.wait()
        return 0

    lax.fori_loop(0, OPS2_TILE // CH2, stage, 0)
    plsc.subcore_barrier()
    _write_acc_cols(out_hbm, c, wb_v, acc_sh, s)


@functools.partial(
    pl.kernel,
    out_type=jax.ShapeDtypeStruct((NPAD, 64), jnp.float32),
    mesh=_mesh,
    compiler_params=_sc_params,
    scratch_types=[
        pltpu.VMEM((CH1, OPW), jnp.int32),
        pltpu.VMEM((CH1, OPW), jnp.int32),
        [pltpu.VMEM((OPW, 16), jnp.float32)] * NB,
        [pltpu.SemaphoreType.DMA] * NB,
        [pltpu.SemaphoreType.DMA] * NB,
        pltpu.VMEM_SHARED((NPAD, 16), jnp.float32),
    ],
)
def _agg1_kernel(srcq_hbm, dst_hbm, tab_hbm, const_hbm, out_hbm,
                 sidx_v, didx_v, bufs, gsems, ssems, acc_sh):
    c = lax.axis_index("c")
    s = lax.axis_index("s")
    for p in range(2):
        q = 2 * c + p
        _zero_acc(const_hbm, bufs[0], acc_sh, s)
        plsc.subcore_barrier()

        def stage(st, _):
            row0 = s * OPS1_TILE + st * CH1
            pltpu.sync_copy(srcq_hbm.at[q, pl.ds(row0, CH1)], sidx_v)
            pltpu.sync_copy(dst_hbm.at[pl.ds(row0, CH1)], didx_v)
            _ring(tab_hbm, acc_sh, sidx_v, didx_v, bufs, gsems, ssems, CH1)
            return 0

        lax.fori_loop(0, OPS1_TILE // CH1, stage, 0)
        plsc.subcore_barrier()
        _write_acc_cols(out_hbm, q, bufs[0], acc_sh, s)
        if p == 0:
            plsc.subcore_barrier()


@functools.partial(
    pl.kernel,
    out_type=jax.ShapeDtypeStruct((NPAD, 32), jnp.float32),
    mesh=_mesh,
    compiler_params=_sc_params,
    scratch_types=[
        pltpu.VMEM((CH2, OPW), jnp.int32),
        pltpu.VMEM((CH2, OPW), jnp.int32),
        [pltpu.VMEM((OPW, 16), jnp.float32)] * NB,
        [pltpu.SemaphoreType.DMA] * NB,
        [pltpu.SemaphoreType.DMA] * NB,
        pltpu.VMEM_SHARED((NPAD, 16), jnp.float32),
    ],
)
def _agg2_kernel(src_hbm, dst_hbm, tab_hbm, const_hbm, out_hbm,
                 sidx_v, didx_v, bufs, gsems, ssems, acc_sh):
    c = lax.axis_index("c")
    s = lax.axis_index("s")
    _zero_acc(const_hbm, bufs[0], acc_sh, s)
    plsc.subcore_barrier()

    def stage(st, _):
        row0 = c * (TOT_OPS // NC) + s * OPS2_TILE + st * CH2
        pltpu.sync_copy(src_hbm.at[pl.ds(row0, CH2)], sidx_v)
        pltpu.sync_copy(dst_hbm.at[pl.ds(row0, CH2)], didx_v)
        _ring(tab_hbm, acc_sh, sidx_v, didx_v, bufs, gsems, ssems, CH2)
        return 0

    lax.fori_loop(0, OPS2_TILE // CH2, stage, 0)
    plsc.subcore_barrier()
    _write_acc_cols(out_hbm, c, bufs[0], acc_sh, s)


def _dense0_body(dacc_ref, x_ref, y_ref, dinv_ref):
    d = dacc_ref[...]
    deg = d[:, 0:1] + d[:, 16:17] + 1.0
    dinv = lax.rsqrt(deg)
    y_ref[...] = x_ref[...] * dinv
    dinv_ref[...] = dinv


def _dense1_body(z_ref, y_ref, dinv_ref, w1_ref, b1_ref, w2_ref, s_ref):
    agg = (z_ref[...] + y_ref[...]) * dinv_ref[...]
    h = jnp.dot(agg, w1_ref[...], preferred_element_type=jnp.float32)
    h = jnp.maximum(h + b1_ref[...], 0.0)
    sp = jnp.dot(h, w2_ref[...], preferred_element_type=jnp.float32)
    sp = sp * dinv_ref[...]
    s_ref[...] = jnp.concatenate(
        [sp, jnp.zeros((sp.shape[0], 15), jnp.float32)], axis=1)


def _dense2_body(z2acc_ref, s16_ref, dinv_ref, b2_ref, out_ref):
    d = z2acc_ref[...]
    z2 = d[:, 0:1] + d[:, 16:17]
    out_ref[...] = dinv_ref[...] * (z2 + s16_ref[...][:, 0:1]) + b2_ref[...]


def kernel(x, edge_index, W1, b1, W2, b2):
    f32 = jnp.float32
    src = edge_index[0].astype(jnp.int32)
    dst = edge_index[1].astype(jnp.int32)
    pad = PAD_E - E
    spread = jnp.arange(pad, dtype=jnp.int32)
    srcp = jnp.concatenate([src, spread % N])
    dstp = jnp.concatenate([dst, N + spread % NGARB])
    src_ops = srcp.reshape(TOT_OPS, OPW)
    dst_ops = dstp.reshape(TOT_OPS, OPW)
    srcq4 = jnp.stack([4 * src_ops + q for q in range(4)])
    x64 = jnp.pad(x, ((0, 0), (0, 64 - IN_DIM)))
    const16 = jnp.stack([jnp.zeros((128, 16), f32), jnp.ones((128, 16), f32)])
    w1p = jnp.pad(W1, ((0, 64 - IN_DIM), (0, 0)))
    b1r = b1.reshape(1, HID)
    b2r = b2.reshape(1, 1)

    dacc = _deg_kernel(dst_ops, const16)

    r0 = 2000
    y64, dinv = pl.pallas_call(
        _dense0_body,
        grid=(N // r0,),
        in_specs=[
            pl.BlockSpec((r0, 32), lambda i: (i, 0)),
            pl.BlockSpec((r0, 64), lambda i: (i, 0)),
        ],
        out_specs=[
            pl.BlockSpec((r0, 64), lambda i: (i, 0)),
            pl.BlockSpec((r0, 1), lambda i: (i, 0)),
        ],
        out_shape=[
            jax.ShapeDtypeStruct((N, 64), f32),
            jax.ShapeDtypeStruct((N, 1), f32),
        ],
    )(dacc, x64)

    zacc = _agg1_kernel(srcq4, dst_ops, y64.reshape(4 * N, 16), const16)

    r1 = 2000
    s16 = pl.pallas_call(
        _dense1_body,
        grid=(N // r1,),
        in_specs=[
            pl.BlockSpec((r1, 64), lambda i: (i, 0)),
            pl.BlockSpec((r1, 64), lambda i: (i, 0)),
            pl.BlockSpec((r1, 1), lambda i: (i, 0)),
            pl.BlockSpec((64, HID), lambda i: (0, 0)),
            pl.BlockSpec((1, HID), lambda i: (0, 0)),
            pl.BlockSpec((HID, 1), lambda i: (0, 0)),
        ],
        out_specs=pl.BlockSpec((r1, 16), lambda i: (i, 0)),
        out_shape=jax.ShapeDtypeStruct((N, 16), f32),
    )(zacc, y64, dinv, w1p, b1r, W2)

    z2acc = _agg2_kernel(src_ops, dst_ops, s16, const16)

    r2 = 2000
    out = pl.pallas_call(
        _dense2_body,
        grid=(N // r2,),
        in_specs=[
            pl.BlockSpec((r2, 32), lambda i: (i, 0)),
            pl.BlockSpec((r2, 16), lambda i: (i, 0)),
            pl.BlockSpec((r2, 1), lambda i: (i, 0)),
            pl.BlockSpec((1, 1), lambda i: (0, 0)),
        ],
        out_specs=pl.BlockSpec((r2, 1), lambda i: (i, 0)),
        out_shape=jax.ShapeDtypeStruct((N, 1), f32),
    )(z2acc, s16, dinv, b2r)
    return out


# speedup vs baseline: 2.0232x; 1.0237x over previous
import functools

import jax
import jax.numpy as jnp
from jax import lax
from jax.experimental import pallas as pl
from jax.experimental.pallas import tpu as pltpu
from jax.experimental.pallas import tpu_sc as plsc

N = 50000
E = 800000
IN_DIM = 58
HID = 100

NC = 2
NS = 16
OPW = 128

PAD_E = 819200
TOT_OPS = PAD_E // OPW
OPS1_TILE = TOT_OPS // NS
OPS2_TILE = TOT_OPS // (NC * NS)
CH1 = 80
CH2 = 40
NB = 10

NPAD = 52096
ROWS_TILE = NPAD // NS
NGARB = NPAD - N

_mesh = plsc.VectorSubcoreMesh(core_axis_name="c", subcore_axis_name="s")
_sc_params = pltpu.CompilerParams(use_tc_tiling_on_sc=False)


def _zero_acc(const_hbm, wb_v, acc_sh, s):
    pltpu.sync_copy(const_hbm.at[0], wb_v)

    def body(k, _):
        r0 = s * ROWS_TILE + k * 128
        pltpu.sync_copy(wb_v, acc_sh.at[pl.ds(r0, 128)])
        return 0

    lax.fori_loop(0, ROWS_TILE // 128, body, 0)
    tail = s * ROWS_TILE + (ROWS_TILE // 128) * 128
    pltpu.sync_copy(wb_v.at[pl.ds(0, ROWS_TILE % 128)],
                    acc_sh.at[pl.ds(tail, ROWS_TILE % 128)])


def _write_acc_cols(out_hbm, q, wb_v, acc_sh, s):
    def body(k, _):
        r0 = s * ROWS_TILE + k * 128
        pltpu.sync_copy(acc_sh.at[pl.ds(r0, 128)], wb_v)
        pltpu.sync_copy(wb_v, out_hbm.at[pl.ds(r0, 128), pl.ds(16 * q, 16)])
        return 0

    lax.fori_loop(0, ROWS_TILE // 128, body, 0)
    tail = s * ROWS_TILE + (ROWS_TILE // 128) * 128
    nt = ROWS_TILE % 128
    pltpu.sync_copy(acc_sh.at[pl.ds(tail, nt)], wb_v.at[pl.ds(0, nt)])
    pltpu.sync_copy(wb_v.at[pl.ds(0, nt)],
                    out_hbm.at[pl.ds(tail, nt), pl.ds(16 * q, 16)])


def _ring(tab_hbm, acc_sh, sidx_v, didx_v, bufs, gsems, ssems, n_ops):
    grp = n_ops // NB
    gd = [pltpu.async_copy(tab_hbm.at[sidx_v.at[b]], bufs[b], gsems[b])
          for b in range(NB)]
    sd = [None] * NB
    for g in range(grp):
        for b in range(NB):
            gd[b].wait()
            sd[b] = pltpu.async_copy(
                bufs[b], acc_sh.at[didx_v.at[g * NB + b]], ssems[b], add=True)
        for b in range(NB):
            sd[b].wait()
            if g + 1 < grp:
                gd[b] = pltpu.async_copy(
                    tab_hbm.at[sidx_v.at[(g + 1) * NB + b]], bufs[b], gsems[b])


@functools.partial(
    pl.kernel,
    out_type=jax.ShapeDtypeStruct((NPAD, 32), jnp.float32),
    mesh=_mesh,
    compiler_params=_sc_params,
    scratch_types=[
        pltpu.VMEM((CH2, OPW), jnp.int32),
        pltpu.VMEM((OPW, 16), jnp.float32),
        pltpu.VMEM((128, 16), jnp.float32),
        [pltpu.SemaphoreType.DMA] * NB,
        pltpu.VMEM_SHARED((NPAD, 16), jnp.float32),
    ],
)
def _deg_kernel(dst_hbm, const_hbm, out_hbm, idx_v, ones_v, wb_v, sems, acc_sh):
    c = lax.axis_index("c")
    s = lax.axis_index("s")
    pltpu.sync_copy(const_hbm.at[1], ones_v)
    _zero_acc(const_hbm, wb_v, acc_sh, s)
    plsc.subcore_barrier()

    def stage(st, _):
        row0 = c * (TOT_OPS // NC) + s * OPS2_TILE + st * CH2
        pltpu.sync_copy(dst_hbm.at[pl.ds(row0, CH2)], idx_v)
        for g in range(CH2 // NB):
            descs = []
            for b in range(NB):
                descs.append(pltpu.async_copy(
                    ones_v, acc_sh.at[idx_v.at[g * NB + b]], sems[b],
                    add=True))
            for d in descs:
                d.wait()
        return 0

    lax.fori_loop(0, OPS2_TILE // CH2, stage, 0)
    plsc.subcore_barrier()
    _write_acc_cols(out_hbm, c, wb_v, acc_sh, s)


@functools.partial(
    pl.kernel,
    out_type=jax.ShapeDtypeStruct((NPAD, 64), jnp.float32),
    mesh=_mesh,
    compiler_params=_sc_params,
    scratch_types=[
        pltpu.VMEM((CH1, OPW), jnp.int32),
        pltpu.VMEM((CH1, OPW), jnp.int32),
        [pltpu.VMEM((OPW, 16), jnp.float32)] * NB,
        [pltpu.SemaphoreType.DMA] * NB,
        [pltpu.SemaphoreType.DMA] * NB,
        pltpu.VMEM_SHARED((NPAD, 16), jnp.float32),
    ],
)
def _agg1_kernel(srcq_hbm, dst_hbm, tab_hbm, const_hbm, out_hbm,
                 sidx_v, didx_v, bufs, gsems, ssems, acc_sh):
    c = lax.axis_index("c")
    s = lax.axis_index("s")
    for p in range(2):
        q = 2 * c + p
        _zero_acc(const_hbm, bufs[0], acc_sh, s)
        plsc.subcore_barrier()

        def stage(st, _):
            row0 = s * OPS1_TILE + st * CH1
            pltpu.sync_copy(srcq_hbm.at[q, pl.ds(row0, CH1)], sidx_v)
            pltpu.sync_copy(dst_hbm.at[pl.ds(row0, CH1)], didx_v)
            _ring(tab_hbm, acc_sh, sidx_v, didx_v, bufs, gsems, ssems, CH1)
            return 0

        lax.fori_loop(0, OPS1_TILE // CH1, stage, 0)
        plsc.subcore_barrier()
        _write_acc_cols(out_hbm, q, bufs[0], acc_sh, s)
        if p == 0:
            plsc.subcore_barrier()


@functools.partial(
    pl.kernel,
    out_type=jax.ShapeDtypeStruct((NPAD, 32), jnp.float32),
    mesh=_mesh,
    compiler_params=_sc_params,
    scratch_types=[
        pltpu.VMEM((CH2, OPW), jnp.int32),
        pltpu.VMEM((CH2, OPW), jnp.int32),
        [pltpu.VMEM((OPW, 16), jnp.float32)] * NB,
        [pltpu.SemaphoreType.DMA] * NB,
        [pltpu.SemaphoreType.DMA] * NB,
        pltpu.VMEM_SHARED((NPAD, 16), jnp.float32),
    ],
)
def _agg2_kernel(src_hbm, dst_hbm, tab_hbm, const_hbm, out_hbm,
                 sidx_v, didx_v, bufs, gsems, ssems, acc_sh):
    c = lax.axis_index("c")
    s = lax.axis_index("s")
    _zero_acc(const_hbm, bufs[0], acc_sh, s)
    plsc.subcore_barrier()

    def stage(st, _):
        row0 = c * (TOT_OPS // NC) + s * OPS2_TILE + st * CH2
        pltpu.sync_copy(src_hbm.at[pl.ds(row0, CH2)], sidx_v)
        pltpu.sync_copy(dst_hbm.at[pl.ds(row0, CH2)], didx_v)
        _ring(tab_hbm, acc_sh, sidx_v, didx_v, bufs, gsems, ssems, CH2)
        return 0

    lax.fori_loop(0, OPS2_TILE // CH2, stage, 0)
    plsc.subcore_barrier()
    _write_acc_cols(out_hbm, c, bufs[0], acc_sh, s)


def _dense0_body(dacc_ref, x_ref, y_ref, dinv_ref):
    d = dacc_ref[...]
    deg = d[:, 0:1] + d[:, 16:17] + 1.0
    dinv = lax.rsqrt(deg)
    y_ref[...] = x_ref[...] * dinv
    dinv_ref[...] = dinv


def _dense1_body(z_ref, y_ref, dinv_ref, w1_ref, b1_ref, w2_ref, s_ref):
    agg = (z_ref[...] + y_ref[...]) * dinv_ref[...]
    h = jnp.dot(agg, w1_ref[...], preferred_element_type=jnp.float32)
    h = jnp.maximum(h + b1_ref[...], 0.0)
    sp = jnp.dot(h, w2_ref[...], preferred_element_type=jnp.float32)
    sp = sp * dinv_ref[...]
    s_ref[...] = jnp.broadcast_to(sp, (sp.shape[0], 16))


def _dense2_body(z2acc_ref, s16_ref, dinv_ref, b2_ref, out_ref):
    d = z2acc_ref[...]
    z2 = d[:, 0:1] + d[:, 16:17]
    out_ref[...] = dinv_ref[...] * (z2 + s16_ref[...][:, 0:1]) + b2_ref[...]


def kernel(x, edge_index, W1, b1, W2, b2):
    f32 = jnp.float32
    src = edge_index[0].astype(jnp.int32)
    dst = edge_index[1].astype(jnp.int32)
    pad = PAD_E - E
    spread = jnp.arange(pad, dtype=jnp.int32)
    srcp = jnp.concatenate([src, spread % N])
    dstp = jnp.concatenate([dst, N + spread % NGARB])
    src_ops = srcp.reshape(TOT_OPS, OPW)
    dst_ops = dstp.reshape(TOT_OPS, OPW)
    srcq4 = jnp.stack([4 * src_ops + q for q in range(4)])
    x64 = jnp.pad(x, ((0, 0), (0, 64 - IN_DIM)))
    const16 = jnp.stack([jnp.zeros((128, 16), f32), jnp.ones((128, 16), f32)])
    w1p = jnp.pad(W1, ((0, 64 - IN_DIM), (0, 0)))
    b1r = b1.reshape(1, HID)
    b2r = b2.reshape(1, 1)

    dacc = _deg_kernel(dst_ops, const16)

    r0 = 5000
    y64, dinv = pl.pallas_call(
        _dense0_body,
        grid=(N // r0,),
        in_specs=[
            pl.BlockSpec((r0, 32), lambda i: (i, 0)),
            pl.BlockSpec((r0, 64), lambda i: (i, 0)),
        ],
        out_specs=[
            pl.BlockSpec((r0, 64), lambda i: (i, 0)),
            pl.BlockSpec((r0, 1), lambda i: (i, 0)),
        ],
        out_shape=[
            jax.ShapeDtypeStruct((N, 64), f32),
            jax.ShapeDtypeStruct((N, 1), f32),
        ],
    )(dacc, x64)

    zacc = _agg1_kernel(srcq4, dst_ops, y64.reshape(4 * N, 16), const16)

    r1 = 5000
    s16 = pl.pallas_call(
        _dense1_body,
        grid=(N // r1,),
        in_specs=[
            pl.BlockSpec((r1, 64), lambda i: (i, 0)),
            pl.BlockSpec((r1, 64), lambda i: (i, 0)),
            pl.BlockSpec((r1, 1), lambda i: (i, 0)),
            pl.BlockSpec((64, HID), lambda i: (0, 0)),
            pl.BlockSpec((1, HID), lambda i: (0, 0)),
            pl.BlockSpec((HID, 1), lambda i: (0, 0)),
        ],
        out_specs=pl.BlockSpec((r1, 16), lambda i: (i, 0)),
        out_shape=jax.ShapeDtypeStruct((N, 16), f32),
    )(zacc, y64, dinv, w1p, b1r, W2)

    z2acc = _agg2_kernel(src_ops, dst_ops, s16, const16)

    r2 = 5000
    out = pl.pallas_call(
        _dense2_body,
        grid=(N // r2,),
        in_specs=[
            pl.BlockSpec((r2, 32), lambda i: (i, 0)),
            pl.BlockSpec((r2, 16), lambda i: (i, 0)),
            pl.BlockSpec((r2, 1), lambda i: (i, 0)),
            pl.BlockSpec((1, 1), lambda i: (0, 0)),
        ],
        out_specs=pl.BlockSpec((r2, 1), lambda i: (i, 0)),
        out_shape=jax.ShapeDtypeStruct((N, 1), f32),
    )(z2acc, s16, dinv, b2r)
    return out
